# baseline (speedup 1.0000x reference)
import numpy as np
import jax
import jax.numpy as jnp
from jax.experimental import pallas as pl
from jax.experimental.pallas import tpu as pltpu

_EPS = 1e-5
_RS = np.float32(1.0 / np.sqrt(1.0 + _EPS))
_DECAY = 0.7

_S = np.zeros((7, 16, 16), np.float32)
for _k in range(7):
    for _t in range(16):
        _d = _t + _k - 3
        if 0 <= _d < 16:
            _S[_k, _d, _t] = 1.0

_pos = np.arange(2048)[:, None].astype(np.float32)
_div = np.exp(np.arange(0, 128, 2).astype(np.float32) * (-np.log(10000.0) / 128))
_PE = np.zeros((2048, 128), np.float32)
_PE[:, 0::2] = np.sin(_pos * _div)
_PE[:, 1::2] = np.cos(_pos * _div)

_tb = np.abs(np.arange(2)[:, None] - np.arange(2)[None, :]).astype(np.float32)
_MASK = np.kron(_DECAY ** _tb, np.ones((32, 32), np.float32))
_EYES = np.tile(np.eye(64, dtype=np.float32), (92, 1))
_MASKS = np.tile(_MASK, (92, 1))

_PERMN = np.concatenate([np.arange(0, 32, 2), np.arange(1, 32, 2)])


def _dot(a, b):
    return jnp.dot(a, b, preferred_element_type=jnp.float32)


def _gram(m):
    return jax.lax.dot_general(m, m, (((1,), (1,)), ((), ())),
                               preferred_element_type=jnp.float32)


_NB = 4


def _main_kernel(x_ref, pe_ref, w1_ref, b1_ref, w2_ref, b2_ref, w3_ref, b3_ref,
                 gc1_ref, gcb1_ref, bs1_ref, bb1_ref, th1_ref, thb1_ref,
                 gc2_ref, gcb2_ref, bs2_ref, bb2_ref, th2_ref, thb2_ref,
                 eyes_ref, masks_ref, f_ref):
    R = 512 * _NB
    x = x_ref[...].reshape(R, 16)
    h = jnp.maximum(_dot(x, w1_ref[...]) + b1_ref[...], 0.0)
    h = jnp.maximum(_dot(h, w2_ref[...]) + b2_ref[...], 0.0)
    a = _dot(h, w3_ref[...]) + b3_ref[...]
    a = (a.reshape(_NB, 16, 32, 128) + pe_ref[...][:, None]).reshape(R, 128)

    map1 = _dot(a, gc1_ref[...]) + gcb1_ref[...]
    map2 = _dot(a, gc2_ref[...]) + gcb2_ref[...]
    xb1 = a * bs1_ref[...] + bb1_ref[...]
    xb2 = a * bs2_ref[...] + bb2_ref[...]

    grams = []
    for bb in range(_NB):
        o = 512 * bb
        grams += [_gram(map1[o + 32 * w:o + 32 * w + 64]) for w in range(15)]
        grams += [_gram(map2[o + 64 * j:o + 64 * j + 64]) for j in range(8)]
    adj = jnp.concatenate(grams, axis=0)

    eye = eyes_ref[...]
    mask = masks_ref[...]
    z = adj - eye * 1e8
    z = jnp.where(z >= 0, z, 0.01 * z)
    z = z - jnp.max(z, axis=-1, keepdims=True)
    e = jnp.exp(z)
    prob = e / jnp.sum(e, axis=-1, keepdims=True)
    adjs = prob * mask + eye

    for bb in range(_NB):
        o = 512 * bb
        q = 1472 * bb
        outs = [_dot(adjs[q + 64 * w:q + 64 * w + 64],
                     xb1[o + 32 * w:o + 32 * w + 64]) for w in range(15)]
        y1 = jnp.concatenate(outs, axis=0)
        outs = [_dot(adjs[q + 960 + 64 * j:q + 1024 + 64 * j],
                     xb2[o + 64 * j:o + 64 * j + 64]) for j in range(8)]
        y2 = jnp.concatenate(outs, axis=0)

        p1 = _dot(y1, th1_ref[...]) + thb1_ref[...]
        p1 = jnp.where(p1 >= 0, p1, 0.01 * p1)
        p2 = _dot(y2, th2_ref[...]) + thb2_ref[...]
        p2 = jnp.where(p2 >= 0, p2, 0.01 * p2)
        p4 = p1.reshape(15, 2, 2, 16, 64)
        m3 = (p4[:, 0] + p4[:, 1]) * 0.5
        f_ref[bb, :240, :] = jnp.concatenate(
            [m3[:, 0].reshape(240, 64), m3[:, 1].reshape(240, 64)], axis=1)
        p4 = p2.reshape(8, 2, 2, 16, 64)
        m3 = (p4[:, 0] + p4[:, 1]) * 0.5
        f_ref[bb, 240:368, :] = jnp.concatenate(
            [m3[:, 0].reshape(128, 64), m3[:, 1].reshape(128, 64)], axis=1)


def _head_kernel(f_ref, w1_ref, b1_ref, w2_ref, b2_ref, w3_ref, b3_ref,
                 w4_ref, b4_ref, o_ref, acc_ref):
    k = pl.program_id(0)

    @pl.when(k == 0)
    def _():
        acc_ref[...] = jnp.zeros_like(acc_ref)

    acc_ref[...] += jax.lax.dot_general(
        f_ref[...], w1_ref[...], (((1,), (1,)), ((), ())),
        preferred_element_type=jnp.float32)

    @pl.when(k == pl.num_programs(0) - 1)
    def _():
        h = jnp.maximum(acc_ref[...] + b1_ref[...], 0.0)
        h = jnp.maximum(_dot(h, w2_ref[...]) + b2_ref[...], 0.0)
        h = jnp.maximum(_dot(h, w3_ref[...]) + b3_ref[...], 0.0)
        o_ref[...] = _dot(h, w4_ref[...]) + b4_ref[...]


def kernel(X, params):
    p = params
    bs = X.shape[0]
    S = jnp.asarray(_S)

    W1 = jnp.einsum('kdt,ok->dot', S, p['conv1_w'][:, 0, :]).reshape(16, 512)
    s1 = p['bn_c1_g'] * _RS
    W1s = W1 * jnp.repeat(s1, 16)[None, :]
    b1f = jnp.repeat(p['bn_c1_b'], 16)[None, :]

    M2 = jnp.einsum('kdt,oik->idot', S, p['conv2_w']).reshape(512, 256)
    s2 = p['bn_c2_g'] * _RS
    M2s = M2 * jnp.repeat(s2, 16)[None, :]
    b2f = jnp.repeat(p['bn_c2_b'], 16)[None, :]

    s3 = p['bn2_g'] * _RS
    L2 = p['lin2_w'].T * s3[None, :]
    blf = (p['lin2_b'] * s3 + p['bn2_b'])[None, :]

    mp = []
    for i in (1, 2):
        gcT = p['gc%d_w' % i].T
        gcb = p['gc%d_b' % i][None, :]
        bns = (p['bnin%d_g' % i] * _RS)[None, :]
        bnb = p['bnin%d_b' % i][None, :]
        sth = p['bnout%d_g' % i] * _RS
        thT = p['th%d_w' % i].T * sth[None, :]
        thb = (p['th%d_b' % i] * sth + p['bnout%d_b' % i])[None, :]
        mp += [gcT, gcb, bns, bnb, thT, thb]

    Xr = X[:, :, _PERMN, :].reshape(bs, 512, 16)
    pe = jnp.asarray(_PE.reshape(64, 32, 128)[:, _PERMN, :])
    eyes = jnp.asarray(_EYES)
    masks = jnp.asarray(_MASKS)

    full = lambda *shape: pl.BlockSpec(shape, lambda b: (0,) * len(shape))
    f = pl.pallas_call(
        _main_kernel,
        grid=(bs // _NB,),
        in_specs=[
            pl.BlockSpec((_NB, 512, 16), lambda b: (b, 0, 0)),
            pl.BlockSpec((_NB, 32, 128), lambda b: (b, 0, 0)),
            full(16, 512), full(1, 512),
            full(512, 256), full(1, 256),
            full(256, 128), full(1, 128),
            full(128, 128), full(1, 128), full(1, 128), full(1, 128),
            full(128, 64), full(1, 64),
            full(128, 128), full(1, 128), full(1, 128), full(1, 128),
            full(128, 64), full(1, 64),
            full(1472 * _NB, 64), full(1472 * _NB, 64),
        ],
        out_specs=pl.BlockSpec((_NB, 368, 128), lambda b: (b, 0, 0)),
        out_shape=jax.ShapeDtypeStruct((bs, 368, 128), jnp.float32),
        compiler_params=pltpu.CompilerParams(
            dimension_semantics=("parallel",)),
    )(Xr, pe, W1s, b1f, M2s, b2f, L2, blf, *mp, eyes, masks)

    f2 = f.reshape(bs, 368 * 128)

    KC = 2048
    NK = f2.shape[1] // KC
    out = pl.pallas_call(
        _head_kernel,
        grid=(NK,),
        in_specs=[
            pl.BlockSpec((bs, KC), lambda k: (0, k)),
            pl.BlockSpec((128, KC), lambda k: (0, k)),
            pl.BlockSpec((1, 128), lambda k: (0, 0)),
            pl.BlockSpec((128, 128), lambda k: (0, 0)),
            pl.BlockSpec((1, 128), lambda k: (0, 0)),
            pl.BlockSpec((128, 64), lambda k: (0, 0)),
            pl.BlockSpec((1, 64), lambda k: (0, 0)),
            pl.BlockSpec((64, 1), lambda k: (0, 0)),
            pl.BlockSpec((1, 1), lambda k: (0, 0)),
        ],
        out_specs=pl.BlockSpec((bs, 1), lambda k: (0, 0)),
        out_shape=jax.ShapeDtypeStruct((bs, 1), jnp.float32),
        scratch_shapes=[pltpu.VMEM((bs, 128), jnp.float32)],
        compiler_params=pltpu.CompilerParams(
            dimension_semantics=("arbitrary",)),
    )(f2, p['fc1_w'], p['fc1_b'][None, :],
      p['fc2_w'].T, p['fc2_b'][None, :],
      p['fc3_w'].T, p['fc3_b'][None, :],
      p['fc4_w'].T, p['fc4_b'][None, :],
      )
    return out


# speedup vs baseline: 1.0838x; 1.0838x over previous
import numpy as np
import jax
import jax.numpy as jnp
from jax.experimental import pallas as pl
from jax.experimental.pallas import tpu as pltpu

_EPS = 1e-5
_RS = np.float32(1.0 / np.sqrt(1.0 + _EPS))
_DECAY = 0.7

_S = np.zeros((7, 16, 16), np.float32)
for _k in range(7):
    for _t in range(16):
        _d = _t + _k - 3
        if 0 <= _d < 16:
            _S[_k, _d, _t] = 1.0

_pos = np.arange(2048)[:, None].astype(np.float32)
_div = np.exp(np.arange(0, 128, 2).astype(np.float32) * (-np.log(10000.0) / 128))
_PE = np.zeros((2048, 128), np.float32)
_PE[:, 0::2] = np.sin(_pos * _div)
_PE[:, 1::2] = np.cos(_pos * _div)

_tb = np.abs(np.arange(2)[:, None] - np.arange(2)[None, :]).astype(np.float32)
_MASK = np.kron(_DECAY ** _tb, np.ones((32, 32), np.float32))
_EYES = np.tile(np.eye(64, dtype=np.float32), (92, 1))
_MASKS = np.tile(_MASK, (92, 1))
_NOEYE = 1.0 - _EYES


def _dot(a, b):
    return jnp.dot(a, b, preferred_element_type=jnp.float32)


def _dot16(a, b):
    return jnp.dot(a.astype(jnp.bfloat16), b, preferred_element_type=jnp.float32)


def _gram(m):
    return jax.lax.dot_general(m, m, (((1,), (1,)), ((), ())),
                               preferred_element_type=jnp.float32)


_NB = 4


def _main_kernel(x_ref, pe_ref, w1_ref, b1_ref, w2_ref, b2_ref, w3_ref, b3_ref,
                 gc1_ref, gcb1_ref, bs1_ref, bb1_ref, th1_ref, thb1_ref,
                 gc2_ref, gcb2_ref, bs2_ref, bb2_ref, th2_ref, thb2_ref,
                 eyes_ref, masks_ref, noeye_ref, f_ref):
    R = 512 * _NB
    x = x_ref[...].reshape(R, 16)
    h = jnp.maximum(_dot16(x, w1_ref[...]) + b1_ref[...], 0.0)
    h = jnp.maximum(_dot16(h, w2_ref[...]) + b2_ref[...], 0.0)
    a = _dot16(h, w3_ref[...]) + b3_ref[...]
    a = (a.reshape(_NB, 16, 32, 128) + pe_ref[...][:, None]).reshape(R, 128)

    map1 = _dot16(a, gc1_ref[...]) + gcb1_ref[...]
    map2 = _dot16(a, gc2_ref[...]) + gcb2_ref[...]
    xb1 = a * bs1_ref[...] + bb1_ref[...]
    xb2 = a * bs2_ref[...] + bb2_ref[...]

    grams = []
    for bb in range(_NB):
        o = 512 * bb
        grams += [_gram(map1[o + 32 * w:o + 32 * w + 64]) for w in range(15)]
        grams += [_gram(map2[o + 64 * j:o + 64 * j + 64]) for j in range(8)]
    adj = jnp.concatenate(grams, axis=0)

    e = jnp.exp(jnp.maximum(adj, 0.01 * adj))
    ed = e * noeye_ref[...]
    rs = 1.0 / jnp.sum(ed, axis=-1, keepdims=True)
    adjs = ed * (masks_ref[...] * rs) + eyes_ref[...]

    for bb in range(_NB):
        o = 512 * bb
        q = 1472 * bb
        outs = [_dot(adjs[q + 64 * w:q + 64 * w + 64],
                     xb1[o + 32 * w:o + 32 * w + 64]) for w in range(15)]
        y1 = jnp.concatenate(outs, axis=0)
        outs = [_dot(adjs[q + 960 + 64 * j:q + 1024 + 64 * j],
                     xb2[o + 64 * j:o + 64 * j + 64]) for j in range(8)]
        y2 = jnp.concatenate(outs, axis=0)

        p1 = _dot16(y1, th1_ref[...]) + thb1_ref[...]
        p1 = jnp.where(p1 >= 0, p1, 0.01 * p1)
        p2 = _dot16(y2, th2_ref[...]) + thb2_ref[...]
        p2 = jnp.where(p2 >= 0, p2, 0.01 * p2)
        f_ref[bb, :480, :] = (
            p1.reshape(15, 2, 32, 64).mean(axis=1).reshape(480, 64))
        f_ref[bb, 480:736, :] = (
            p2.reshape(8, 2, 32, 64).mean(axis=1).reshape(256, 64))


def _head_kernel(f_ref, w1_ref, b1_ref, w2_ref, b2_ref, w3_ref, b3_ref,
                 w4_ref, b4_ref, o_ref, acc_ref):
    k = pl.program_id(0)

    @pl.when(k == 0)
    def _():
        acc_ref[...] = jnp.zeros_like(acc_ref)

    acc_ref[...] += jax.lax.dot_general(
        f_ref[...], w1_ref[...], (((1,), (1,)), ((), ())),
        preferred_element_type=jnp.float32)

    @pl.when(k == pl.num_programs(0) - 1)
    def _():
        h = jnp.maximum(acc_ref[...] + b1_ref[...], 0.0)
        h = jnp.maximum(_dot(h, w2_ref[...]) + b2_ref[...], 0.0)
        h = jnp.maximum(_dot(h, w3_ref[...]) + b3_ref[...], 0.0)
        o_ref[...] = _dot(h, w4_ref[...]) + b4_ref[...]


def kernel(X, params):
    p = params
    bs = X.shape[0]
    S = jnp.asarray(_S)

    W1 = jnp.einsum('kdt,ok->dot', S, p['conv1_w'][:, 0, :]).reshape(16, 512)
    s1 = p['bn_c1_g'] * _RS
    W1s = W1 * jnp.repeat(s1, 16)[None, :]
    b1f = jnp.repeat(p['bn_c1_b'], 16)[None, :]

    M2 = jnp.einsum('kdt,oik->idot', S, p['conv2_w']).reshape(512, 256)
    s2 = p['bn_c2_g'] * _RS
    M2s = M2 * jnp.repeat(s2, 16)[None, :]
    b2f = jnp.repeat(p['bn_c2_b'], 16)[None, :]

    s3 = p['bn2_g'] * _RS
    L2 = p['lin2_w'].T * s3[None, :]
    blf = (p['lin2_b'] * s3 + p['bn2_b'])[None, :]

    mp = []
    for i in (1, 2):
        gcT = p['gc%d_w' % i].T.astype(jnp.bfloat16)
        gcb = p['gc%d_b' % i][None, :]
        bns = (p['bnin%d_g' % i] * _RS)[None, :]
        bnb = p['bnin%d_b' % i][None, :]
        sth = p['bnout%d_g' % i] * _RS
        thT = (p['th%d_w' % i].T * sth[None, :]).astype(jnp.bfloat16)
        thb = (p['th%d_b' % i] * sth + p['bnout%d_b' % i])[None, :]
        mp += [gcT, gcb, bns, bnb, thT, thb]

    Xr = X.reshape(bs, 512, 16)
    pe = jnp.asarray(_PE).reshape(64, 32, 128)
    eyes = jnp.asarray(_EYES)
    masks = jnp.asarray(_MASKS)

    full = lambda *shape: pl.BlockSpec(shape, lambda b: (0,) * len(shape))
    f = pl.pallas_call(
        _main_kernel,
        grid=(bs // _NB,),
        in_specs=[
            pl.BlockSpec((_NB, 512, 16), lambda b: (b, 0, 0)),
            pl.BlockSpec((_NB, 32, 128), lambda b: (b, 0, 0)),
            full(16, 512), full(1, 512),
            full(512, 256), full(1, 256),
            full(256, 128), full(1, 128),
            full(128, 128), full(1, 128), full(1, 128), full(1, 128),
            full(128, 64), full(1, 64),
            full(128, 128), full(1, 128), full(1, 128), full(1, 128),
            full(128, 64), full(1, 64),
            full(1472 * _NB, 64), full(1472 * _NB, 64), full(1472 * _NB, 64),
        ],
        out_specs=pl.BlockSpec((_NB, 736, 64), lambda b: (b, 0, 0)),
        out_shape=jax.ShapeDtypeStruct((bs, 736, 64), jnp.float32),
        compiler_params=pltpu.CompilerParams(
            dimension_semantics=("parallel",)),
    )(Xr, pe, W1s.astype(jnp.bfloat16), b1f, M2s.astype(jnp.bfloat16), b2f,
      L2.astype(jnp.bfloat16), blf, *mp, eyes, masks, jnp.asarray(_NOEYE))

    f2 = f.reshape(bs, 736 * 64)

    KC = 2048
    NK = f2.shape[1] // KC
    out = pl.pallas_call(
        _head_kernel,
        grid=(NK,),
        in_specs=[
            pl.BlockSpec((bs, KC), lambda k: (0, k)),
            pl.BlockSpec((128, KC), lambda k: (0, k)),
            pl.BlockSpec((1, 128), lambda k: (0, 0)),
            pl.BlockSpec((128, 128), lambda k: (0, 0)),
            pl.BlockSpec((1, 128), lambda k: (0, 0)),
            pl.BlockSpec((128, 64), lambda k: (0, 0)),
            pl.BlockSpec((1, 64), lambda k: (0, 0)),
            pl.BlockSpec((64, 1), lambda k: (0, 0)),
            pl.BlockSpec((1, 1), lambda k: (0, 0)),
        ],
        out_specs=pl.BlockSpec((bs, 1), lambda k: (0, 0)),
        out_shape=jax.ShapeDtypeStruct((bs, 1), jnp.float32),
        scratch_shapes=[pltpu.VMEM((bs, 128), jnp.float32)],
        compiler_params=pltpu.CompilerParams(
            dimension_semantics=("arbitrary",)),
    )(f2, p['fc1_w'], p['fc1_b'][None, :],
      p['fc2_w'].T, p['fc2_b'][None, :],
      p['fc3_w'].T, p['fc3_b'][None, :],
      p['fc4_w'].T, p['fc4_b'][None, :],
      )
    return out


# speedup vs baseline: 1.1320x; 1.0445x over previous
import numpy as np
import jax
import jax.numpy as jnp
from jax.experimental import pallas as pl
from jax.experimental.pallas import tpu as pltpu

_EPS = 1e-5
_RS = np.float32(1.0 / np.sqrt(1.0 + _EPS))
_DECAY = 0.7

_S = np.zeros((7, 16, 16), np.float32)
for _k in range(7):
    for _t in range(16):
        _d = _t + _k - 3
        if 0 <= _d < 16:
            _S[_k, _d, _t] = 1.0

_pos = np.arange(2048)[:, None].astype(np.float32)
_div = np.exp(np.arange(0, 128, 2).astype(np.float32) * (-np.log(10000.0) / 128))
_PE = np.zeros((2048, 128), np.float32)
_PE[:, 0::2] = np.sin(_pos * _div)
_PE[:, 1::2] = np.cos(_pos * _div)

_tb = np.abs(np.arange(2)[:, None] - np.arange(2)[None, :]).astype(np.float32)
_MASK = np.kron(_DECAY ** _tb, np.ones((32, 32), np.float32))
_EYE = np.eye(64, dtype=np.float32)
_NOEYE1 = 1.0 - _EYE


def _dot(a, b):
    return jnp.dot(a, b, preferred_element_type=jnp.float32)


def _dot16(a, b):
    return jnp.dot(a.astype(jnp.bfloat16), b, preferred_element_type=jnp.float32)


def _gram(m):
    return jax.lax.dot_general(m, m, (((1,), (1,)), ((), ())),
                               preferred_element_type=jnp.float32)


_NB = 8


def _main_kernel(x_ref, pe_ref, w1_ref, b1_ref, w2_ref, b2_ref, w3_ref, b3_ref,
                 gc1_ref, gcb1_ref, bs1_ref, bb1_ref, th1_ref, thb1_ref,
                 gc2_ref, gcb2_ref, bs2_ref, bb2_ref, th2_ref, thb2_ref,
                 eyes_ref, masks_ref, noeye_ref, f_ref):
    R = 512 * _NB
    x = x_ref[...].reshape(R, 16)
    h = jnp.maximum(_dot16(x, w1_ref[...]) + b1_ref[...], 0.0)
    h = jnp.maximum(_dot16(h, w2_ref[...]) + b2_ref[...], 0.0)
    a = _dot16(h, w3_ref[...]) + b3_ref[...]
    a = (a.reshape(_NB, 16, 32, 128) + pe_ref[...][:, None]).reshape(R, 128)

    map1 = _dot16(a, gc1_ref[...]) + gcb1_ref[...]
    map2 = _dot16(a, gc2_ref[...]) + gcb2_ref[...]
    xb1 = a * bs1_ref[...] + bb1_ref[...]
    xb2 = a * bs2_ref[...] + bb2_ref[...]

    grams = []
    for bb in range(_NB):
        o = 512 * bb
        grams += [_gram(map1[o + 32 * w:o + 32 * w + 64]) for w in range(15)]
        grams += [_gram(map2[o + 64 * j:o + 64 * j + 64]) for j in range(8)]
    adj = jnp.concatenate(grams, axis=0)

    e = jnp.exp(jnp.maximum(adj, 0.01 * adj))
    e3 = e.reshape(_NB * 23, 64, 64)
    ed = e3 * noeye_ref[...]
    rs = 1.0 / jnp.sum(ed, axis=-1, keepdims=True)
    adjs = (ed * (masks_ref[...] * rs)
            + eyes_ref[...]).reshape(_NB * 1472, 64)

    for bb in range(_NB):
        o = 512 * bb
        q = 1472 * bb
        outs = [_dot(adjs[q + 64 * w:q + 64 * w + 64],
                     xb1[o + 32 * w:o + 32 * w + 64]) for w in range(15)]
        y1 = jnp.concatenate(outs, axis=0)
        outs = [_dot(adjs[q + 960 + 64 * j:q + 1024 + 64 * j],
                     xb2[o + 64 * j:o + 64 * j + 64]) for j in range(8)]
        y2 = jnp.concatenate(outs, axis=0)

        p1 = _dot16(y1, th1_ref[...]) + thb1_ref[...]
        p1 = jnp.where(p1 >= 0, p1, 0.01 * p1)
        p2 = _dot16(y2, th2_ref[...]) + thb2_ref[...]
        p2 = jnp.where(p2 >= 0, p2, 0.01 * p2)
        f_ref[bb, :480, :] = (
            p1.reshape(15, 2, 32, 64).mean(axis=1).reshape(480, 64))
        f_ref[bb, 480:736, :] = (
            p2.reshape(8, 2, 32, 64).mean(axis=1).reshape(256, 64))


def _head_kernel(f_ref, w1_ref, b1_ref, w2_ref, b2_ref, w3_ref, b3_ref,
                 w4_ref, b4_ref, o_ref, acc_ref):
    k = pl.program_id(0)

    @pl.when(k == 0)
    def _():
        acc_ref[...] = jnp.zeros_like(acc_ref)

    acc_ref[...] += jax.lax.dot_general(
        f_ref[...], w1_ref[...], (((1,), (1,)), ((), ())),
        preferred_element_type=jnp.float32)

    @pl.when(k == pl.num_programs(0) - 1)
    def _():
        h = jnp.maximum(acc_ref[...] + b1_ref[...], 0.0)
        h = jnp.maximum(_dot(h, w2_ref[...]) + b2_ref[...], 0.0)
        h = jnp.maximum(_dot(h, w3_ref[...]) + b3_ref[...], 0.0)
        o_ref[...] = _dot(h, w4_ref[...]) + b4_ref[...]


def kernel(X, params):
    p = params
    bs = X.shape[0]
    S = jnp.asarray(_S)

    W1 = jnp.einsum('kdt,ok->dot', S, p['conv1_w'][:, 0, :]).reshape(16, 512)
    s1 = p['bn_c1_g'] * _RS
    W1s = W1 * jnp.repeat(s1, 16)[None, :]
    b1f = jnp.repeat(p['bn_c1_b'], 16)[None, :]

    M2 = jnp.einsum('kdt,oik->idot', S, p['conv2_w']).reshape(512, 256)
    s2 = p['bn_c2_g'] * _RS
    M2s = M2 * jnp.repeat(s2, 16)[None, :]
    b2f = jnp.repeat(p['bn_c2_b'], 16)[None, :]

    s3 = p['bn2_g'] * _RS
    L2 = p['lin2_w'].T * s3[None, :]
    blf = (p['lin2_b'] * s3 + p['bn2_b'])[None, :]

    mp = []
    for i in (1, 2):
        gcT = p['gc%d_w' % i].T.astype(jnp.bfloat16)
        gcb = p['gc%d_b' % i][None, :]
        bns = (p['bnin%d_g' % i] * _RS)[None, :]
        bnb = p['bnin%d_b' % i][None, :]
        sth = p['bnout%d_g' % i] * _RS
        thT = (p['th%d_w' % i].T * sth[None, :]).astype(jnp.bfloat16)
        thb = (p['th%d_b' % i] * sth + p['bnout%d_b' % i])[None, :]
        mp += [gcT, gcb, bns, bnb, thT, thb]

    Xr = X.reshape(bs, 512, 16)
    pe = jnp.asarray(_PE).reshape(64, 32, 128)
    eyes = jnp.asarray(_EYE)
    masks = jnp.asarray(_MASK)

    full = lambda *shape: pl.BlockSpec(shape, lambda b: (0,) * len(shape))
    f = pl.pallas_call(
        _main_kernel,
        grid=(bs // _NB,),
        in_specs=[
            pl.BlockSpec((_NB, 512, 16), lambda b: (b, 0, 0)),
            pl.BlockSpec((_NB, 32, 128), lambda b: (b, 0, 0)),
            full(16, 512), full(1, 512),
            full(512, 256), full(1, 256),
            full(256, 128), full(1, 128),
            full(128, 128), full(1, 128), full(1, 128), full(1, 128),
            full(128, 64), full(1, 64),
            full(128, 128), full(1, 128), full(1, 128), full(1, 128),
            full(128, 64), full(1, 64),
            full(64, 64), full(64, 64), full(64, 64),
        ],
        out_specs=pl.BlockSpec((_NB, 736, 64), lambda b: (b, 0, 0)),
        out_shape=jax.ShapeDtypeStruct((bs, 736, 64), jnp.float32),
        compiler_params=pltpu.CompilerParams(
            dimension_semantics=("parallel",)),
    )(Xr, pe, W1s.astype(jnp.bfloat16), b1f, M2s.astype(jnp.bfloat16), b2f,
      L2.astype(jnp.bfloat16), blf, *mp, eyes, masks, jnp.asarray(_NOEYE1))

    f2 = f.reshape(bs, 736 * 64)

    KC = 2048
    NK = f2.shape[1] // KC
    out = pl.pallas_call(
        _head_kernel,
        grid=(NK,),
        in_specs=[
            pl.BlockSpec((bs, KC), lambda k: (0, k)),
            pl.BlockSpec((128, KC), lambda k: (0, k)),
            pl.BlockSpec((1, 128), lambda k: (0, 0)),
            pl.BlockSpec((128, 128), lambda k: (0, 0)),
            pl.BlockSpec((1, 128), lambda k: (0, 0)),
            pl.BlockSpec((128, 64), lambda k: (0, 0)),
            pl.BlockSpec((1, 64), lambda k: (0, 0)),
            pl.BlockSpec((64, 1), lambda k: (0, 0)),
            pl.BlockSpec((1, 1), lambda k: (0, 0)),
        ],
        out_specs=pl.BlockSpec((bs, 1), lambda k: (0, 0)),
        out_shape=jax.ShapeDtypeStruct((bs, 1), jnp.float32),
        scratch_shapes=[pltpu.VMEM((bs, 128), jnp.float32)],
        compiler_params=pltpu.CompilerParams(
            dimension_semantics=("arbitrary",)),
    )(f2, p['fc1_w'], p['fc1_b'][None, :],
      p['fc2_w'].T, p['fc2_b'][None, :],
      p['fc3_w'].T, p['fc3_b'][None, :],
      p['fc4_w'].T, p['fc4_b'][None, :],
      )
    return out


# speedup vs baseline: 1.1366x; 1.0041x over previous
import numpy as np
import jax
import jax.numpy as jnp
from jax.experimental import pallas as pl
from jax.experimental.pallas import tpu as pltpu

_EPS = 1e-5
_RS = np.float32(1.0 / np.sqrt(1.0 + _EPS))
_DECAY = 0.7

_S = np.zeros((7, 16, 16), np.float32)
for _k in range(7):
    for _t in range(16):
        _d = _t + _k - 3
        if 0 <= _d < 16:
            _S[_k, _d, _t] = 1.0

_pos = np.arange(2048)[:, None].astype(np.float32)
_div = np.exp(np.arange(0, 128, 2).astype(np.float32) * (-np.log(10000.0) / 128))
_PE = np.zeros((2048, 128), np.float32)
_PE[:, 0::2] = np.sin(_pos * _div)
_PE[:, 1::2] = np.cos(_pos * _div)

_tb = np.abs(np.arange(2)[:, None] - np.arange(2)[None, :]).astype(np.float32)
_MASK = np.kron(_DECAY ** _tb, np.ones((32, 32), np.float32))
_EYE = np.eye(64, dtype=np.float32)
_NOEYE1 = 1.0 - _EYE


def _dot(a, b):
    return jnp.dot(a, b, preferred_element_type=jnp.float32)


def _gram(m):
    return jax.lax.dot_general(m, m, (((1,), (1,)), ((), ())),
                               preferred_element_type=jnp.float32)


_NB = 8


def _main_kernel(x_ref, pe_ref, w1_ref, b1_ref, w2_ref, b2_ref, w3_ref, b3_ref,
                 gc1_ref, gcb1_ref, bs1_ref, bb1_ref, th1_ref, thb1_ref,
                 gc2_ref, gcb2_ref, bs2_ref, bb2_ref, th2_ref, thb2_ref,
                 eyes_ref, masks_ref, noeye_ref, f_ref):
    R = 512 * _NB
    x = x_ref[...].reshape(R, 16)
    h = jnp.maximum(_dot(x, w1_ref[...]) + b1_ref[...], 0.0)
    h = jnp.maximum(_dot(h, w2_ref[...]) + b2_ref[...], 0.0)
    a = _dot(h, w3_ref[...]) + b3_ref[...]
    a = (a.reshape(_NB, 16, 32, 128) + pe_ref[...][:, None]).reshape(R, 128)

    map1 = _dot(a, gc1_ref[...]) + gcb1_ref[...]
    map2 = _dot(a, gc2_ref[...]) + gcb2_ref[...]
    xb1 = a * bs1_ref[...] + bb1_ref[...]
    xb2 = a * bs2_ref[...] + bb2_ref[...]

    grams = []
    for bb in range(_NB):
        o = 512 * bb
        grams += [_gram(map1[o + 32 * w:o + 32 * w + 64]) for w in range(15)]
        grams += [_gram(map2[o + 64 * j:o + 64 * j + 64]) for j in range(8)]
    adj = jnp.concatenate(grams, axis=0)

    e = jnp.exp(jnp.maximum(adj, 0.01 * adj))
    e3 = e.reshape(_NB * 23, 64, 64)
    ed = e3 * noeye_ref[...]
    rs = 1.0 / jnp.sum(ed, axis=-1, keepdims=True)
    adjs = (ed * (masks_ref[...] * rs)
            + eyes_ref[...]).reshape(_NB * 1472, 64)

    for bb in range(_NB):
        o = 512 * bb
        q = 1472 * bb
        outs = [_dot(adjs[q + 64 * w:q + 64 * w + 64],
                     xb1[o + 32 * w:o + 32 * w + 64]) for w in range(15)]
        y1 = jnp.concatenate(outs, axis=0)
        outs = [_dot(adjs[q + 960 + 64 * j:q + 1024 + 64 * j],
                     xb2[o + 64 * j:o + 64 * j + 64]) for j in range(8)]
        y2 = jnp.concatenate(outs, axis=0)

        p1 = _dot(y1, th1_ref[...]) + thb1_ref[...]
        p1 = jnp.where(p1 >= 0, p1, 0.01 * p1)
        p2 = _dot(y2, th2_ref[...]) + thb2_ref[...]
        p2 = jnp.where(p2 >= 0, p2, 0.01 * p2)
        f_ref[bb, :480, :] = (
            p1.reshape(15, 2, 32, 64).mean(axis=1).reshape(480, 64))
        f_ref[bb, 480:736, :] = (
            p2.reshape(8, 2, 32, 64).mean(axis=1).reshape(256, 64))


def _head_kernel(f_ref, w1_ref, b1_ref, w2_ref, b2_ref, w3_ref, b3_ref,
                 w4_ref, b4_ref, o_ref, acc_ref):
    k = pl.program_id(0)

    @pl.when(k == 0)
    def _():
        acc_ref[...] = jnp.zeros_like(acc_ref)

    acc_ref[...] += jax.lax.dot_general(
        f_ref[...], w1_ref[...], (((1,), (1,)), ((), ())),
        preferred_element_type=jnp.float32)

    @pl.when(k == pl.num_programs(0) - 1)
    def _():
        h = jnp.maximum(acc_ref[...] + b1_ref[...], 0.0)
        h = jnp.maximum(_dot(h, w2_ref[...]) + b2_ref[...], 0.0)
        h = jnp.maximum(_dot(h, w3_ref[...]) + b3_ref[...], 0.0)
        o_ref[...] = _dot(h, w4_ref[...]) + b4_ref[...]


def kernel(X, params):
    p = params
    bs = X.shape[0]
    S = jnp.asarray(_S)

    W1 = jnp.einsum('kdt,ok->dot', S, p['conv1_w'][:, 0, :]).reshape(16, 512)
    s1 = p['bn_c1_g'] * _RS
    W1s = W1 * jnp.repeat(s1, 16)[None, :]
    b1f = jnp.repeat(p['bn_c1_b'], 16)[None, :]

    M2 = jnp.einsum('kdt,oik->idot', S, p['conv2_w']).reshape(512, 256)
    s2 = p['bn_c2_g'] * _RS
    M2s = M2 * jnp.repeat(s2, 16)[None, :]
    b2f = jnp.repeat(p['bn_c2_b'], 16)[None, :]

    s3 = p['bn2_g'] * _RS
    L2 = p['lin2_w'].T * s3[None, :]
    blf = (p['lin2_b'] * s3 + p['bn2_b'])[None, :]

    mp = []
    for i in (1, 2):
        gcT = p['gc%d_w' % i].T
        gcb = p['gc%d_b' % i][None, :]
        bns = (p['bnin%d_g' % i] * _RS)[None, :]
        bnb = p['bnin%d_b' % i][None, :]
        sth = p['bnout%d_g' % i] * _RS
        thT = p['th%d_w' % i].T * sth[None, :]
        thb = (p['th%d_b' % i] * sth + p['bnout%d_b' % i])[None, :]
        mp += [gcT, gcb, bns, bnb, thT, thb]

    Xr = X.reshape(bs, 512, 16)
    pe = jnp.asarray(_PE).reshape(64, 32, 128)
    eyes = jnp.asarray(_EYE)
    masks = jnp.asarray(_MASK)

    full = lambda *shape: pl.BlockSpec(shape, lambda b: (0,) * len(shape))
    f = pl.pallas_call(
        _main_kernel,
        grid=(bs // _NB,),
        in_specs=[
            pl.BlockSpec((_NB, 512, 16), lambda b: (b, 0, 0)),
            pl.BlockSpec((_NB, 32, 128), lambda b: (b, 0, 0)),
            full(16, 512), full(1, 512),
            full(512, 256), full(1, 256),
            full(256, 128), full(1, 128),
            full(128, 128), full(1, 128), full(1, 128), full(1, 128),
            full(128, 64), full(1, 64),
            full(128, 128), full(1, 128), full(1, 128), full(1, 128),
            full(128, 64), full(1, 64),
            full(64, 64), full(64, 64), full(64, 64),
        ],
        out_specs=pl.BlockSpec((_NB, 736, 64), lambda b: (b, 0, 0)),
        out_shape=jax.ShapeDtypeStruct((bs, 736, 64), jnp.float32),
        compiler_params=pltpu.CompilerParams(
            dimension_semantics=("parallel",)),
    )(Xr, pe, W1s, b1f, M2s, b2f,
      L2, blf, *mp, eyes, masks, jnp.asarray(_NOEYE1))

    f2 = f.reshape(bs, 736 * 64)

    KC = 2048
    NK = f2.shape[1] // KC
    out = pl.pallas_call(
        _head_kernel,
        grid=(NK,),
        in_specs=[
            pl.BlockSpec((bs, KC), lambda k: (0, k)),
            pl.BlockSpec((128, KC), lambda k: (0, k)),
            pl.BlockSpec((1, 128), lambda k: (0, 0)),
            pl.BlockSpec((128, 128), lambda k: (0, 0)),
            pl.BlockSpec((1, 128), lambda k: (0, 0)),
            pl.BlockSpec((128, 64), lambda k: (0, 0)),
            pl.BlockSpec((1, 64), lambda k: (0, 0)),
            pl.BlockSpec((64, 1), lambda k: (0, 0)),
            pl.BlockSpec((1, 1), lambda k: (0, 0)),
        ],
        out_specs=pl.BlockSpec((bs, 1), lambda k: (0, 0)),
        out_shape=jax.ShapeDtypeStruct((bs, 1), jnp.float32),
        scratch_shapes=[pltpu.VMEM((bs, 128), jnp.float32)],
        compiler_params=pltpu.CompilerParams(
            dimension_semantics=("arbitrary",)),
    )(f2, p['fc1_w'], p['fc1_b'][None, :],
      p['fc2_w'].T, p['fc2_b'][None, :],
      p['fc3_w'].T, p['fc3_b'][None, :],
      p['fc4_w'].T, p['fc4_b'][None, :],
      )
    return out
